# baseline (speedup 1.0000x reference)
import numpy as np
import jax
import jax.numpy as jnp
from jax import lax
from jax.experimental import pallas as pl
from jax.experimental.pallas import tpu as pltpu

_ATAN_C = (0.9999955125536241, -0.33298872907314714, 0.1955895093384993,
           -0.12111029635484719, 0.05733117442459627, -0.013422329575982384)
_COS_C = (0.999999953271256, -0.49999905044325044, 0.04166357820492561,
          -0.001385366054692888, 2.3153014743704237e-05)
_SIN_C = (0.9999992416158777, -0.16665679451349188, 0.008313221654705674,
          -0.00018523321129623738)

_PI_HI = 3.1415927
_PIO2 = 1.5707964
_INV_PI = 0.31830987


def _watan_sincos(x, w):
    ax = jnp.abs(x)
    big = ax > 1.0
    z = jnp.where(big, 1.0 / ax, ax)
    z2 = z * z
    p = jnp.float32(_ATAN_C[5])
    for k in (4, 3, 2, 1, 0):
        p = p * z2 + jnp.float32(_ATAN_C[k])
    th = p * z
    th = jnp.where(big, _PIO2 - th, th)
    sbit = lax.bitcast_convert_type(x, jnp.int32) & jnp.int32(-2147483648)
    th = lax.bitcast_convert_type(
        lax.bitcast_convert_type(th, jnp.int32) | sbit, jnp.float32)

    a = th * w
    t = a * _INV_PI
    mf = jnp.round(t)
    sgn = (mf.astype(jnp.int32) & 1) << 31
    r = a - mf * _PI_HI
    r2 = r * r
    pc = jnp.float32(_COS_C[4])
    for k in (3, 2, 1, 0):
        pc = pc * r2 + jnp.float32(_COS_C[k])
    ps = jnp.float32(_SIN_C[3])
    for k in (2, 1, 0):
        ps = ps * r2 + jnp.float32(_SIN_C[k])
    ps = ps * r
    c = lax.bitcast_convert_type(
        lax.bitcast_convert_type(pc, jnp.int32) ^ sgn, jnp.float32)
    s = lax.bitcast_convert_type(
        lax.bitcast_convert_type(ps, jnp.int32) ^ sgn, jnp.float32)
    return c, s

_NQ = 4
_NA = 2
_SUB = 16
_LANE = 128
_CHUNKS = 32
_STEP_ROWS = _SUB * _CHUNKS
_STEP_OBS = _STEP_ROWS * _LANE

_T0 = [
    (+1, 'X', 'X', 'YYZZ'),
    (-1, 'X', 'Y', 'YXZZ'),
    (+1, 'X', 'Z', 'XIIZ'),
    (-1, 'Y', 'X', 'XYZZ'),
    (+1, 'Y', 'Y', 'XXZZ'),
    (+1, 'Y', 'Z', 'YIIZ'),
    (+1, 'Z', 'X', 'IXZI'),
    (+1, 'Z', 'Y', 'IYZI'),
    (+1, 'Z', 'Z', 'ZZII'),
]
def _build_deint():
    S = np.zeros((2 * _NQ * _LANE, _NQ * _LANE), np.float32)
    for j in range(_LANE):
        for w in range(_NQ):
            S[4 * j + w, _LANE * w + j] = 1.0
            S[_NQ * _LANE + 4 * j + w, _LANE * w + j] = 1.0
    return S.astype(jnp.bfloat16)


def _build_int():
    T = np.zeros((2 * 2 * _LANE, 2 * _LANE), np.float32)
    for k in range(_LANE):
        for half in (0, 2 * _LANE):
            T[half + k, 2 * k] = 1.0
            T[half + _LANE + k, 2 * k + 1] = 1.0
    return T.astype(jnp.bfloat16)


_T1 = [
    (+1, 'X', 'X', 'ZZYY'),
    (-1, 'X', 'Y', 'ZZYX'),
    (+1, 'X', 'Z', 'IZXI'),
    (-1, 'Y', 'X', 'ZZXY'),
    (+1, 'Y', 'Y', 'ZZXX'),
    (+1, 'Y', 'Z', 'IZYI'),
    (+1, 'Z', 'X', 'ZIIX'),
    (+1, 'Z', 'Y', 'ZIIY'),
    (+1, 'Z', 'Z', 'IIZZ'),
]


def _eval_pair(A, B, zAB, zCD, cBhD, cAhC, c, s, h, K):
    hXA, hXB = h[A]['X'], h[B]['X']
    hYA, hYB = h[A]['Y'], h[B]['Y']
    pA = s[A] * hXA
    pB = s[B] * hXB
    g1 = (K[0] * (hYA * hYB) + K[1] * (hYA * pB)
          + K[3] * (pA * hYB) + K[4] * (pA * pB)) * zCD
    g2 = (K[2] * hXA + K[5] * (s[A] * hYA)) * cBhD
    g3 = (K[6] * hXB + K[7] * (s[B] * hYB)) * cAhC
    g4 = K[8] * (c[A] * c[B]) * zAB
    return g1 + g2 + g3 + g4


def _hi_lo_cat(v):
    hi = v.astype(jnp.bfloat16)
    lo = (v - hi.astype(jnp.float32)).astype(jnp.bfloat16)
    return jnp.concatenate([hi, lo], axis=1)


def _kernel_body(x_ref, s_ref, t_ref, sc_ref, coef_ref, out_ref,
                 pln_ref, oacc_ref):
    pln_ref[...] = jnp.dot(_hi_lo_cat(x_ref[...]), s_ref[...],
                           preferred_element_type=jnp.float32)
    for jj in range(_CHUNKS):
        base = jj * _SUB
        c, s, h = [None] * _NQ, [None] * _NQ, [None] * _NQ
        for w in range(_NQ):
            xw = pln_ref[pl.ds(base, _SUB), w * _LANE:(w + 1) * _LANE]
            cw, sw = _watan_sincos(xw, sc_ref[0, w])
            c[w], s[w] = cw, sw
            h[w] = {
                'X': sw * sc_ref[1, w] + cw * sc_ref[2, w],
                'Y': sw * sc_ref[3, w] + cw * sc_ref[4, w],
                'Z': cw * sc_ref[5, w],
            }
        z01 = h[0]['Z'] * h[1]['Z']
        z23 = h[2]['Z'] * h[3]['Z']
        K0 = [coef_ref[i] for i in range(9)]
        K1 = [coef_ref[9 + i] for i in range(9)]
        e0 = _eval_pair(0, 1, z01, z23, c[1] * h[3]['Z'], c[0] * h[2]['Z'],
                        c, s, h, K0)
        e1 = _eval_pair(2, 3, z23, z01, c[3] * h[1]['Z'], c[2] * h[0]['Z'],
                        c, s, h, K1)
        oacc_ref[pl.ds(base, _SUB), :] = jnp.concatenate(
            [sc_ref[6, 0] * (1.0 + e0),
             sc_ref[6, 1] * (1.0 + e1)], axis=1)
    out_ref[...] = jnp.dot(_hi_lo_cat(oacc_ref[...]), t_ref[...],
                           preferred_element_type=jnp.float32)


def _forward(x, w_input, y_weights, z_weights, w_output):
    n = x.shape[0]
    nsteps = -(-n // _STEP_OBS)
    npad = nsteps * _STEP_OBS
    xp = x.astype(jnp.float32)
    if npad != n:
        xp = jnp.zeros((npad, _NQ), jnp.float32).at[:n].set(xp)
    rows = npad // _LANE
    xr = xp.reshape(rows, _NQ * _LANE)

    y1, y2 = y_weights[0], y_weights[1]
    z1 = z_weights[0]
    sy1, cy1 = jnp.sin(y1), jnp.cos(y1)
    sz1, cz1 = jnp.sin(z1), jnp.cos(z1)
    sy2, cy2 = jnp.sin(y2), jnp.cos(y2)

    sc = jnp.stack([
        w_input.astype(jnp.float32),
        sz1, sy1 * cz1,
        -cz1, sy1 * sz1,
        cy1,
        jnp.concatenate([0.5 * w_output.astype(jnp.float32),
                         jnp.zeros((_NQ - _NA,), jnp.float32)]),
    ]).astype(jnp.float32)

    def vscal(i, w):
        return -sy2[w] if i == 'X' else cy2[w]

    coefs = []
    for terms, (wa, wb) in ((_T0, (0, 1)), (_T1, (2, 3))):
        for sgn, i, j, _ in terms:
            coefs.append(sgn * vscal(i, wa) * vscal(j, wb))
    coef = jnp.stack(coefs).astype(jnp.float32)

    out = pl.pallas_call(
        _kernel_body,
        out_shape=jax.ShapeDtypeStruct((rows, _NA * _LANE), jnp.float32),
        grid=(nsteps,),
        in_specs=[
            pl.BlockSpec((_STEP_ROWS, _NQ * _LANE), lambda i: (i, 0)),
            pl.BlockSpec((2 * _NQ * _LANE, _NQ * _LANE), lambda i: (0, 0)),
            pl.BlockSpec((2 * _NA * _LANE, _NA * _LANE), lambda i: (0, 0)),
            pl.BlockSpec(memory_space=pltpu.MemorySpace.SMEM),
            pl.BlockSpec(memory_space=pltpu.MemorySpace.SMEM),
        ],
        out_specs=pl.BlockSpec((_STEP_ROWS, _NA * _LANE), lambda i: (i, 0)),
        scratch_shapes=[
            pltpu.VMEM((_STEP_ROWS, _NQ * _LANE), jnp.float32),
            pltpu.VMEM((_STEP_ROWS, _NA * _LANE), jnp.float32),
        ],
        compiler_params=pltpu.CompilerParams(dimension_semantics=("parallel",)),
    )(xr, _build_deint(), _build_int(), sc, coef)

    res = out.reshape(npad, _NA)
    return res[:n]


def kernel(x, w_input, y_weights, z_weights, w_output):
    if x.ndim == 1:
        return _forward(x[None, :], w_input, y_weights, z_weights, w_output)[0]
    return _forward(x, w_input, y_weights, z_weights, w_output)


# speedup vs baseline: 2.8236x; 2.8236x over previous
import numpy as np
import jax
import jax.numpy as jnp
from jax import lax
from jax.experimental import pallas as pl
from jax.experimental.pallas import tpu as pltpu

_ATAN_C = (0.9999955125536241, -0.33298872907314714, 0.1955895093384993,
           -0.12111029635484719, 0.05733117442459627, -0.013422329575982384)
_COS_C = (0.999999953271256, -0.49999905044325044, 0.04166357820492561,
          -0.001385366054692888, 2.3153014743704237e-05)
_SIN_C = (0.9999992416158777, -0.16665679451349188, 0.008313221654705674,
          -0.00018523321129623738)

_PI_HI = 3.1415927
_PIO2 = 1.5707964
_INV_PI = 0.31830987


def _watan_sincos(x, w):
    ax = jnp.abs(x)
    big = ax > 1.0
    z = jnp.where(big, 1.0 / ax, ax)
    z2 = z * z
    p = jnp.float32(_ATAN_C[5])
    for k in (4, 3, 2, 1, 0):
        p = p * z2 + jnp.float32(_ATAN_C[k])
    th = p * z
    th = jnp.where(big, _PIO2 - th, th)
    sbit = lax.bitcast_convert_type(x, jnp.int32) & jnp.int32(-2147483648)
    th = lax.bitcast_convert_type(
        lax.bitcast_convert_type(th, jnp.int32) | sbit, jnp.float32)

    a = th * w
    t = a * _INV_PI
    mf = jnp.round(t)
    sgn = (mf.astype(jnp.int32) & 1) << 31
    r = a - mf * _PI_HI
    r2 = r * r
    pc = jnp.float32(_COS_C[4])
    for k in (3, 2, 1, 0):
        pc = pc * r2 + jnp.float32(_COS_C[k])
    ps = jnp.float32(_SIN_C[3])
    for k in (2, 1, 0):
        ps = ps * r2 + jnp.float32(_SIN_C[k])
    ps = ps * r
    c = lax.bitcast_convert_type(
        lax.bitcast_convert_type(pc, jnp.int32) ^ sgn, jnp.float32)
    s = lax.bitcast_convert_type(
        lax.bitcast_convert_type(ps, jnp.int32) ^ sgn, jnp.float32)
    return c, s

_NQ = 4
_NA = 2
_SUB = 16
_LANE = 128
_CHUNKS = 32
_STEP_ROWS = _SUB * _CHUNKS
_STEP_OBS = _STEP_ROWS * _LANE

_T0 = [
    (+1, 'X', 'X', 'YYZZ'),
    (-1, 'X', 'Y', 'YXZZ'),
    (+1, 'X', 'Z', 'XIIZ'),
    (-1, 'Y', 'X', 'XYZZ'),
    (+1, 'Y', 'Y', 'XXZZ'),
    (+1, 'Y', 'Z', 'YIIZ'),
    (+1, 'Z', 'X', 'IXZI'),
    (+1, 'Z', 'Y', 'IYZI'),
    (+1, 'Z', 'Z', 'ZZII'),
]
def _build_deint():
    S = np.zeros((2 * _NQ * _LANE, _NQ * _LANE), np.float32)
    for j in range(_LANE):
        for w in range(_NQ):
            S[4 * j + w, _LANE * w + j] = 1.0
            S[_NQ * _LANE + 4 * j + w, _LANE * w + j] = 1.0
    return S.astype(jnp.bfloat16)


def _build_int():
    T = np.zeros((2 * 2 * _LANE, 2 * _LANE), np.float32)
    for k in range(_LANE):
        for half in (0, 2 * _LANE):
            T[half + k, 2 * k] = 1.0
            T[half + _LANE + k, 2 * k + 1] = 1.0
    return T.astype(jnp.bfloat16)


_T1 = [
    (+1, 'X', 'X', 'ZZYY'),
    (-1, 'X', 'Y', 'ZZYX'),
    (+1, 'X', 'Z', 'IZXI'),
    (-1, 'Y', 'X', 'ZZXY'),
    (+1, 'Y', 'Y', 'ZZXX'),
    (+1, 'Y', 'Z', 'IZYI'),
    (+1, 'Z', 'X', 'ZIIX'),
    (+1, 'Z', 'Y', 'ZIIY'),
    (+1, 'Z', 'Z', 'IIZZ'),
]


def _eval_pair(A, B, zAB, zCD, cBhD, cAhC, c, s, h, K):
    hXA, hXB = h[A]['X'], h[B]['X']
    hYA, hYB = h[A]['Y'], h[B]['Y']
    pA = s[A] * hXA
    pB = s[B] * hXB
    g1 = (K[0] * (hYA * hYB) + K[1] * (hYA * pB)
          + K[3] * (pA * hYB) + K[4] * (pA * pB)) * zCD
    g2 = (K[2] * hXA + K[5] * (s[A] * hYA)) * cBhD
    g3 = (K[6] * hXB + K[7] * (s[B] * hYB)) * cAhC
    g4 = K[8] * (c[A] * c[B]) * zAB
    return g1 + g2 + g3 + g4


def _hi_lo_cat(v):
    hi = v.astype(jnp.bfloat16)
    lo = (v - hi.astype(jnp.float32)).astype(jnp.bfloat16)
    return jnp.concatenate([hi, lo], axis=1)


def _kernel_body(x_ref, t_ref, sc_ref, coef_ref, out_ref, oacc_ref):
    for jj in range(_CHUNKS):
        base = jj * _SUB
        c, s, h = [None] * _NQ, [None] * _NQ, [None] * _NQ
        for w in range(_NQ):
            xw = x_ref[w, pl.ds(base, _SUB), :]
            cw, sw = _watan_sincos(xw, sc_ref[0, w])
            c[w], s[w] = cw, sw
            h[w] = {
                'X': sw * sc_ref[1, w] + cw * sc_ref[2, w],
                'Y': sw * sc_ref[3, w] + cw * sc_ref[4, w],
                'Z': cw * sc_ref[5, w],
            }
        z01 = h[0]['Z'] * h[1]['Z']
        z23 = h[2]['Z'] * h[3]['Z']
        K0 = [coef_ref[i] for i in range(9)]
        K1 = [coef_ref[9 + i] for i in range(9)]
        e0 = _eval_pair(0, 1, z01, z23, c[1] * h[3]['Z'], c[0] * h[2]['Z'],
                        c, s, h, K0)
        e1 = _eval_pair(2, 3, z23, z01, c[3] * h[1]['Z'], c[2] * h[0]['Z'],
                        c, s, h, K1)
        oacc_ref[pl.ds(base, _SUB), :] = jnp.concatenate(
            [sc_ref[6, 0] * (1.0 + e0),
             sc_ref[6, 1] * (1.0 + e1)], axis=1)
    out_ref[...] = jnp.dot(_hi_lo_cat(oacc_ref[...]), t_ref[...],
                           preferred_element_type=jnp.float32)


def _forward(x, w_input, y_weights, z_weights, w_output):
    n = x.shape[0]
    nsteps = -(-n // _STEP_OBS)
    npad = nsteps * _STEP_OBS
    xp = x.astype(jnp.float32)
    if npad != n:
        xp = jnp.zeros((npad, _NQ), jnp.float32).at[:n].set(xp)
    rows = npad // _LANE
    xt = xp.T.reshape(_NQ, rows, _LANE)

    y1, y2 = y_weights[0], y_weights[1]
    z1 = z_weights[0]
    sy1, cy1 = jnp.sin(y1), jnp.cos(y1)
    sz1, cz1 = jnp.sin(z1), jnp.cos(z1)
    sy2, cy2 = jnp.sin(y2), jnp.cos(y2)

    sc = jnp.stack([
        w_input.astype(jnp.float32),
        sz1, sy1 * cz1,
        -cz1, sy1 * sz1,
        cy1,
        jnp.concatenate([0.5 * w_output.astype(jnp.float32),
                         jnp.zeros((_NQ - _NA,), jnp.float32)]),
    ]).astype(jnp.float32)

    def vscal(i, w):
        return -sy2[w] if i == 'X' else cy2[w]

    coefs = []
    for terms, (wa, wb) in ((_T0, (0, 1)), (_T1, (2, 3))):
        for sgn, i, j, _ in terms:
            coefs.append(sgn * vscal(i, wa) * vscal(j, wb))
    coef = jnp.stack(coefs).astype(jnp.float32)

    out = pl.pallas_call(
        _kernel_body,
        out_shape=jax.ShapeDtypeStruct((rows, _NA * _LANE), jnp.float32),
        grid=(nsteps,),
        in_specs=[
            pl.BlockSpec((_NQ, _STEP_ROWS, _LANE), lambda i: (0, i, 0)),
            pl.BlockSpec((2 * _NA * _LANE, _NA * _LANE), lambda i: (0, 0)),
            pl.BlockSpec(memory_space=pltpu.MemorySpace.SMEM),
            pl.BlockSpec(memory_space=pltpu.MemorySpace.SMEM),
        ],
        out_specs=pl.BlockSpec((_STEP_ROWS, _NA * _LANE), lambda i: (i, 0)),
        scratch_shapes=[
            pltpu.VMEM((_STEP_ROWS, _NA * _LANE), jnp.float32),
        ],
        compiler_params=pltpu.CompilerParams(dimension_semantics=("parallel",)),
    )(xt, _build_int(), sc, coef)

    res = out.reshape(npad, _NA)
    return res[:n]


def kernel(x, w_input, y_weights, z_weights, w_output):
    if x.ndim == 1:
        return _forward(x[None, :], w_input, y_weights, z_weights, w_output)[0]
    return _forward(x, w_input, y_weights, z_weights, w_output)


# speedup vs baseline: 20.0899x; 7.1151x over previous
import jax
import jax.numpy as jnp
from jax import lax
from jax.experimental import pallas as pl
from jax.experimental.pallas import tpu as pltpu

_COS_C = (0.999999953271256, -0.49999905044325044, 0.04166357820492561,
          -0.001385366054692888, 2.3153014743704237e-05)
_SIN_C = (0.9999992416158777, -0.16665679451349188, 0.008313221654705674,
          -0.00018523321129623738)

_PI_HI = 3.1415927
_INV_PI = 0.31830987


def _sincos(a):
    t = a * _INV_PI
    mf = jnp.round(t)
    sgn = (mf.astype(jnp.int32) & 1) << 31
    r = a - mf * _PI_HI
    r2 = r * r
    pc = jnp.float32(_COS_C[4])
    for k in (3, 2, 1, 0):
        pc = pc * r2 + jnp.float32(_COS_C[k])
    ps = jnp.float32(_SIN_C[3])
    for k in (2, 1, 0):
        ps = ps * r2 + jnp.float32(_SIN_C[k])
    ps = ps * r
    c = lax.bitcast_convert_type(
        lax.bitcast_convert_type(pc, jnp.int32) ^ sgn, jnp.float32)
    s = lax.bitcast_convert_type(
        lax.bitcast_convert_type(ps, jnp.int32) ^ sgn, jnp.float32)
    return c, s

_NQ = 4
_NA = 2
_SUB = 16
_LANE = 128
_CHUNKS = 32
_STEP_ROWS = _SUB * _CHUNKS
_STEP_OBS = _STEP_ROWS * _LANE

_T0 = [
    (+1, 'X', 'X', 'YYZZ'),
    (-1, 'X', 'Y', 'YXZZ'),
    (+1, 'X', 'Z', 'XIIZ'),
    (-1, 'Y', 'X', 'XYZZ'),
    (+1, 'Y', 'Y', 'XXZZ'),
    (+1, 'Y', 'Z', 'YIIZ'),
    (+1, 'Z', 'X', 'IXZI'),
    (+1, 'Z', 'Y', 'IYZI'),
    (+1, 'Z', 'Z', 'ZZII'),
]
_T1 = [
    (+1, 'X', 'X', 'ZZYY'),
    (-1, 'X', 'Y', 'ZZYX'),
    (+1, 'X', 'Z', 'IZXI'),
    (-1, 'Y', 'X', 'ZZXY'),
    (+1, 'Y', 'Y', 'ZZXX'),
    (+1, 'Y', 'Z', 'IZYI'),
    (+1, 'Z', 'X', 'ZIIX'),
    (+1, 'Z', 'Y', 'ZIIY'),
    (+1, 'Z', 'Z', 'IIZZ'),
]


def _eval_pair(A, B, zAB, zCD, cBhD, cAhC, c, s, h, K):
    hXA, hXB = h[A]['X'], h[B]['X']
    hYA, hYB = h[A]['Y'], h[B]['Y']
    pA = s[A] * hXA
    pB = s[B] * hXB
    g1 = (K[0] * (hYA * hYB) + K[1] * (hYA * pB)
          + K[3] * (pA * hYB) + K[4] * (pA * pB)) * zCD
    g2 = (K[2] * hXA + K[5] * (s[A] * hYA)) * cBhD
    g3 = (K[6] * hXB + K[7] * (s[B] * hYB)) * cAhC
    g4 = K[8] * (c[A] * c[B]) * zAB
    return g1 + g2 + g3 + g4


def _kernel_body(x_ref, sc_ref, coef_ref, out_ref):
    for jj in range(_CHUNKS):
        base = jj * _SUB
        c, s, h = [None] * _NQ, [None] * _NQ, [None] * _NQ
        for w in range(_NQ):
            aw = x_ref[w, pl.ds(base, _SUB), :]
            cw, sw = _sincos(aw)
            c[w], s[w] = cw, sw
            h[w] = {
                'X': sw * sc_ref[0, w] + cw * sc_ref[1, w],
                'Y': sw * sc_ref[2, w] + cw * sc_ref[3, w],
                'Z': cw * sc_ref[4, w],
            }
        z01 = h[0]['Z'] * h[1]['Z']
        z23 = h[2]['Z'] * h[3]['Z']
        K0 = [coef_ref[i] for i in range(9)]
        K1 = [coef_ref[9 + i] for i in range(9)]
        e0 = _eval_pair(0, 1, z01, z23, c[1] * h[3]['Z'], c[0] * h[2]['Z'],
                        c, s, h, K0)
        e1 = _eval_pair(2, 3, z23, z01, c[3] * h[1]['Z'], c[2] * h[0]['Z'],
                        c, s, h, K1)
        out_ref[0, pl.ds(base, _SUB), :] = sc_ref[5, 0] * (1.0 + e0)
        out_ref[1, pl.ds(base, _SUB), :] = sc_ref[5, 1] * (1.0 + e1)


def _forward(x, w_input, y_weights, z_weights, w_output):
    n = x.shape[0]
    nsteps = -(-n // _STEP_OBS)
    npad = nsteps * _STEP_OBS
    xp = x.astype(jnp.float32)
    if npad != n:
        xp = jnp.zeros((npad, _NQ), jnp.float32).at[:n].set(xp)
    rows = npad // _LANE
    at = (jnp.arctan(xp) * w_input.astype(jnp.float32)).T.reshape(
        _NQ, rows, _LANE)

    y1, y2 = y_weights[0], y_weights[1]
    z1 = z_weights[0]
    sy1, cy1 = jnp.sin(y1), jnp.cos(y1)
    sz1, cz1 = jnp.sin(z1), jnp.cos(z1)
    sy2, cy2 = jnp.sin(y2), jnp.cos(y2)

    sc = jnp.stack([
        sz1, sy1 * cz1,
        -cz1, sy1 * sz1,
        cy1,
        jnp.concatenate([0.5 * w_output.astype(jnp.float32),
                         jnp.zeros((_NQ - _NA,), jnp.float32)]),
    ]).astype(jnp.float32)

    def vscal(i, w):
        return -sy2[w] if i == 'X' else cy2[w]

    coefs = []
    for terms, (wa, wb) in ((_T0, (0, 1)), (_T1, (2, 3))):
        for sgn, i, j, _ in terms:
            coefs.append(sgn * vscal(i, wa) * vscal(j, wb))
    coef = jnp.stack(coefs).astype(jnp.float32)

    out = pl.pallas_call(
        _kernel_body,
        out_shape=jax.ShapeDtypeStruct((_NA, rows, _LANE), jnp.float32),
        grid=(nsteps,),
        in_specs=[
            pl.BlockSpec((_NQ, _STEP_ROWS, _LANE), lambda i: (0, i, 0)),
            pl.BlockSpec(memory_space=pltpu.MemorySpace.SMEM),
            pl.BlockSpec(memory_space=pltpu.MemorySpace.SMEM),
        ],
        out_specs=pl.BlockSpec((_NA, _STEP_ROWS, _LANE), lambda i: (0, i, 0)),
        compiler_params=pltpu.CompilerParams(dimension_semantics=("parallel",)),
    )(at, sc, coef)

    res = out.reshape(_NA, npad).T
    return res[:n]


def kernel(x, w_input, y_weights, z_weights, w_output):
    if x.ndim == 1:
        return _forward(x[None, :], w_input, y_weights, z_weights, w_output)[0]
    return _forward(x, w_input, y_weights, z_weights, w_output)


# speedup vs baseline: 25.2879x; 1.2587x over previous
import jax
import jax.numpy as jnp
from jax import lax
from jax.experimental import pallas as pl
from jax.experimental.pallas import tpu as pltpu

_ATAN_C = (0.9999955125536241, -0.33298872907314714, 0.1955895093384993,
           -0.12111029635484719, 0.05733117442459627, -0.013422329575982384)
_COS_C = (0.999999953271256, -0.49999905044325044, 0.04166357820492561,
          -0.001385366054692888, 2.3153014743704237e-05)
_SIN_C = (0.9999992416158777, -0.16665679451349188, 0.008313221654705674,
          -0.00018523321129623738)

_PI_HI = 3.1415927
_PIO2 = 1.5707964
_INV_PI = 0.31830987


def _watan_sincos(x, w):
    ax = jnp.abs(x)
    big = ax > 1.0
    z = jnp.where(big, 1.0 / ax, ax)
    z2 = z * z
    p = jnp.float32(_ATAN_C[5])
    for k in (4, 3, 2, 1, 0):
        p = p * z2 + jnp.float32(_ATAN_C[k])
    th = p * z
    th = jnp.where(big, _PIO2 - th, th)
    sbit = lax.bitcast_convert_type(x, jnp.int32) & jnp.int32(-2147483648)
    th = lax.bitcast_convert_type(
        lax.bitcast_convert_type(th, jnp.int32) | sbit, jnp.float32)

    a = th * w
    t = a * _INV_PI
    mf = jnp.round(t)
    sgn = (mf.astype(jnp.int32) & 1) << 31
    r = a - mf * _PI_HI
    r2 = r * r
    pc = jnp.float32(_COS_C[4])
    for k in (3, 2, 1, 0):
        pc = pc * r2 + jnp.float32(_COS_C[k])
    ps = jnp.float32(_SIN_C[3])
    for k in (2, 1, 0):
        ps = ps * r2 + jnp.float32(_SIN_C[k])
    ps = ps * r
    c = lax.bitcast_convert_type(
        lax.bitcast_convert_type(pc, jnp.int32) ^ sgn, jnp.float32)
    s = lax.bitcast_convert_type(
        lax.bitcast_convert_type(ps, jnp.int32) ^ sgn, jnp.float32)
    return c, s

_NQ = 4
_NA = 2
_SUB = 16
_LANE = 128
_CHUNKS = 32
_STEP_ROWS = _SUB * _CHUNKS
_STEP_OBS = _STEP_ROWS * _LANE

_T0 = [
    (+1, 'X', 'X', 'YYZZ'),
    (-1, 'X', 'Y', 'YXZZ'),
    (+1, 'X', 'Z', 'XIIZ'),
    (-1, 'Y', 'X', 'XYZZ'),
    (+1, 'Y', 'Y', 'XXZZ'),
    (+1, 'Y', 'Z', 'YIIZ'),
    (+1, 'Z', 'X', 'IXZI'),
    (+1, 'Z', 'Y', 'IYZI'),
    (+1, 'Z', 'Z', 'ZZII'),
]
_T1 = [
    (+1, 'X', 'X', 'ZZYY'),
    (-1, 'X', 'Y', 'ZZYX'),
    (+1, 'X', 'Z', 'IZXI'),
    (-1, 'Y', 'X', 'ZZXY'),
    (+1, 'Y', 'Y', 'ZZXX'),
    (+1, 'Y', 'Z', 'IZYI'),
    (+1, 'Z', 'X', 'ZIIX'),
    (+1, 'Z', 'Y', 'ZIIY'),
    (+1, 'Z', 'Z', 'IIZZ'),
]


def _eval_pair(A, B, zAB, zCD, cBhD, cAhC, c, s, h, K):
    hXA, hXB = h[A]['X'], h[B]['X']
    hYA, hYB = h[A]['Y'], h[B]['Y']
    pA = s[A] * hXA
    pB = s[B] * hXB
    g1 = (K[0] * (hYA * hYB) + K[1] * (hYA * pB)
          + K[3] * (pA * hYB) + K[4] * (pA * pB)) * zCD
    g2 = (K[2] * hXA + K[5] * (s[A] * hYA)) * cBhD
    g3 = (K[6] * hXB + K[7] * (s[B] * hYB)) * cAhC
    g4 = K[8] * (c[A] * c[B]) * zAB
    return g1 + g2 + g3 + g4


def _kernel_body(x_ref, sc_ref, coef_ref, out_ref):
    for jj in range(_CHUNKS):
        base = jj * _SUB
        c, s, h = [None] * _NQ, [None] * _NQ, [None] * _NQ
        for w in range(_NQ):
            xw = x_ref[w, pl.ds(base, _SUB), :]
            cw, sw = _watan_sincos(xw, sc_ref[0, w])
            c[w], s[w] = cw, sw
            h[w] = {
                'X': sw * sc_ref[1, w] + cw * sc_ref[2, w],
                'Y': sw * sc_ref[3, w] + cw * sc_ref[4, w],
                'Z': cw * sc_ref[5, w],
            }
        z01 = h[0]['Z'] * h[1]['Z']
        z23 = h[2]['Z'] * h[3]['Z']
        K0 = [coef_ref[i] for i in range(9)]
        K1 = [coef_ref[9 + i] for i in range(9)]
        e0 = _eval_pair(0, 1, z01, z23, c[1] * h[3]['Z'], c[0] * h[2]['Z'],
                        c, s, h, K0)
        e1 = _eval_pair(2, 3, z23, z01, c[3] * h[1]['Z'], c[2] * h[0]['Z'],
                        c, s, h, K1)
        out_ref[0, pl.ds(base, _SUB), :] = sc_ref[6, 0] * (1.0 + e0)
        out_ref[1, pl.ds(base, _SUB), :] = sc_ref[6, 1] * (1.0 + e1)


def _forward(x, w_input, y_weights, z_weights, w_output):
    n = x.shape[0]
    nsteps = -(-n // _STEP_OBS)
    npad = nsteps * _STEP_OBS
    xp = x.astype(jnp.float32)
    if npad != n:
        xp = jnp.zeros((npad, _NQ), jnp.float32).at[:n].set(xp)
    rows = npad // _LANE
    xt = xp.T.reshape(_NQ, rows, _LANE)

    y1, y2 = y_weights[0], y_weights[1]
    z1 = z_weights[0]
    sy1, cy1 = jnp.sin(y1), jnp.cos(y1)
    sz1, cz1 = jnp.sin(z1), jnp.cos(z1)
    sy2, cy2 = jnp.sin(y2), jnp.cos(y2)

    sc = jnp.stack([
        w_input.astype(jnp.float32),
        sz1, sy1 * cz1,
        -cz1, sy1 * sz1,
        cy1,
        jnp.concatenate([0.5 * w_output.astype(jnp.float32),
                         jnp.zeros((_NQ - _NA,), jnp.float32)]),
    ]).astype(jnp.float32)

    def vscal(i, w):
        return -sy2[w] if i == 'X' else cy2[w]

    coefs = []
    for terms, (wa, wb) in ((_T0, (0, 1)), (_T1, (2, 3))):
        for sgn, i, j, _ in terms:
            coefs.append(sgn * vscal(i, wa) * vscal(j, wb))
    coef = jnp.stack(coefs).astype(jnp.float32)

    out = pl.pallas_call(
        _kernel_body,
        out_shape=jax.ShapeDtypeStruct((_NA, rows, _LANE), jnp.float32),
        grid=(nsteps,),
        in_specs=[
            pl.BlockSpec((_NQ, _STEP_ROWS, _LANE), lambda i: (0, i, 0)),
            pl.BlockSpec(memory_space=pltpu.MemorySpace.SMEM),
            pl.BlockSpec(memory_space=pltpu.MemorySpace.SMEM),
        ],
        out_specs=pl.BlockSpec((_NA, _STEP_ROWS, _LANE), lambda i: (0, i, 0)),
        compiler_params=pltpu.CompilerParams(dimension_semantics=("parallel",)),
    )(xt, sc, coef)

    res = out.reshape(_NA, npad).T
    return res[:n]


def kernel(x, w_input, y_weights, z_weights, w_output):
    if x.ndim == 1:
        return _forward(x[None, :], w_input, y_weights, z_weights, w_output)[0]
    return _forward(x, w_input, y_weights, z_weights, w_output)
